# baseline (speedup 1.0000x reference)
import functools

import jax
import jax.numpy as jnp
from jax import lax
from jax.experimental import pallas as pl
from jax.experimental.pallas import tpu as pltpu
from jax.experimental.pallas import tpu_sc as plsc

VOCAB = 100000
EMBD = 64
B = 1024

NC = 2
NS = 16
NW = NC * NS
BPW = B // NW

TILE_B = 4096


def _bridge_body(tt_ref, out_ref):
    t = tt_ref[...]
    tt = lax.dot_general(
        t, jnp.eye(EMBD, dtype=jnp.float32),
        (((0,), (0,)), ((), ())),
        preferred_element_type=jnp.float32,
    )
    out_ref[...] = jnp.concatenate(
        [tt, jnp.zeros((TILE_B, 128 - EMBD), jnp.float32)], axis=1
    )


def _tc_bridge(tablet):
    return pl.pallas_call(
        _bridge_body,
        grid=(pl.cdiv(VOCAB, TILE_B),),
        in_specs=[pl.BlockSpec((EMBD, TILE_B), lambda i: (0, i))],
        out_specs=pl.BlockSpec((TILE_B, 128), lambda i: (i, 0)),
        out_shape=jax.ShapeDtypeStruct((VOCAB, 128), jnp.float32),
        compiler_params=pltpu.CompilerParams(
            dimension_semantics=("arbitrary",),
        ),
    )(tablet)


def _gather_body(tablep_hbm, idx_hbm, out_hbm, idx_v, rows_v, sem):
    wid = lax.axis_index("s") * NC + lax.axis_index("c")
    base = wid * BPW
    pltpu.sync_copy(idx_hbm.at[pl.ds(base, BPW)], idx_v)
    pltpu.async_copy(tablep_hbm.at[idx_v], rows_v, sem).wait()
    pltpu.sync_copy(rows_v, out_hbm.at[pl.ds(base, BPW)])


def _sc_gather(tablep, idx):
    mesh = plsc.VectorSubcoreMesh(core_axis_name="c", subcore_axis_name="s")
    return pl.kernel(
        _gather_body,
        mesh=mesh,
        out_type=jax.ShapeDtypeStruct((B, 128), jnp.float32),
        scratch_types=[
            pltpu.VMEM((BPW,), jnp.int32),
            pltpu.VMEM((BPW, 128), jnp.float32),
            pltpu.SemaphoreType.DMA,
        ],
        compiler_params=pltpu.CompilerParams(use_tc_tiling_on_sc=True),
    )(tablep, idx)


TILE_V = 1024
NSTEP = pl.cdiv(VOCAB, TILE_V)
TAIL_V = VOCAB - (NSTEP - 1) * TILE_V
NBUF = 4


def _mm_body(emb2_ref, w_ref, b_ref, out_hbm, scratch, sems):
    i = pl.program_id(0)
    buf = lax.rem(i, NBUF)

    @pl.when(i >= NBUF)
    def _():
        pltpu.make_async_copy(
            scratch.at[buf],
            out_hbm.at[pl.ds((i - NBUF) * TILE_V, TILE_V)],
            sems.at[buf],
        ).wait()

    w = w_ref[...].astype(jnp.bfloat16)
    emb = emb2_ref[:, :EMBD].astype(jnp.bfloat16)
    acc = lax.dot_general(
        w, emb, (((0,), (1,)), ((), ())),
        preferred_element_type=jnp.float32,
    )
    bcol = lax.dot_general(
        b_ref[...],
        jnp.ones((1, 1), jnp.float32),
        (((0,), (1,)), ((), ())),
        preferred_element_type=jnp.float32,
    )
    scratch[buf] = acc + bcol

    @pl.when(i < NSTEP - 1)
    def _():
        pltpu.make_async_copy(
            scratch.at[buf],
            out_hbm.at[pl.ds(i * TILE_V, TILE_V)],
            sems.at[buf],
        ).start()

    @pl.when(i == NSTEP - 1)
    def _():
        pltpu.make_async_copy(
            scratch.at[buf, pl.ds(0, TAIL_V)],
            out_hbm.at[pl.ds(i * TILE_V, TAIL_V)],
            sems.at[buf],
        ).start()
        for k in range(NBUF):
            step = NSTEP - NBUF + k
            kbuf = step % NBUF
            if step == NSTEP - 1:
                pltpu.make_async_copy(
                    scratch.at[kbuf, pl.ds(0, TAIL_V)],
                    out_hbm.at[pl.ds(step * TILE_V, TAIL_V)],
                    sems.at[kbuf],
                ).wait()
            else:
                pltpu.make_async_copy(
                    scratch.at[kbuf],
                    out_hbm.at[pl.ds(step * TILE_V, TILE_V)],
                    sems.at[kbuf],
                ).wait()


def _tc_matmul_t(emb2, W, brow):
    return pl.pallas_call(
        _mm_body,
        grid=(NSTEP,),
        in_specs=[
            pl.BlockSpec((B, 128), lambda i: (0, 0)),
            pl.BlockSpec((EMBD, TILE_V), lambda i: (0, i)),
            pl.BlockSpec((1, TILE_V), lambda i: (0, i)),
        ],
        out_specs=pl.BlockSpec(memory_space=pl.ANY),
        out_shape=jax.ShapeDtypeStruct((VOCAB, B), jnp.float32),
        scratch_shapes=[
            pltpu.VMEM((NBUF, TILE_V, B), jnp.float32),
            pltpu.SemaphoreType.DMA((NBUF,)),
        ],
        compiler_params=pltpu.CompilerParams(
            dimension_semantics=("arbitrary",),
        ),
    )(emb2, W, brow)


@jax.jit
def kernel(idx, table, W, b):
    idx32 = idx.astype(jnp.int32)
    tablep = _tc_bridge(table.T)
    emb2 = _sc_gather(tablep, idx32)
    out_t = _tc_matmul_t(emb2, W, b.reshape(1, VOCAB))
    return out_t.T


# speedup vs baseline: 1.0628x; 1.0628x over previous
import functools

import jax
import jax.numpy as jnp
from jax import lax
from jax.experimental import pallas as pl
from jax.experimental.pallas import tpu as pltpu
from jax.experimental.pallas import tpu_sc as plsc

VOCAB = 100000
EMBD = 64
B = 1024

NC = 2
NS = 16
NW = NC * NS
BPW = B // NW

TILE_B = 4096


def _bridge_body(tt_ref, out_ref):
    t = tt_ref[...]
    tt = lax.dot_general(
        t, jnp.eye(EMBD, dtype=jnp.float32),
        (((0,), (0,)), ((), ())),
        preferred_element_type=jnp.float32,
    )
    out_ref[...] = jnp.concatenate(
        [tt, jnp.zeros((TILE_B, 128 - EMBD), jnp.float32)], axis=1
    )


def _tc_bridge(tablet):
    return pl.pallas_call(
        _bridge_body,
        grid=(pl.cdiv(VOCAB, TILE_B),),
        in_specs=[pl.BlockSpec((EMBD, TILE_B), lambda i: (0, i))],
        out_specs=pl.BlockSpec((TILE_B, 128), lambda i: (i, 0)),
        out_shape=jax.ShapeDtypeStruct((VOCAB, 128), jnp.float32),
        compiler_params=pltpu.CompilerParams(
            dimension_semantics=("arbitrary",),
        ),
    )(tablet)


def _gather_body(tablep_hbm, idx_hbm, out_hbm, idx_v, rows_v, sem):
    wid = lax.axis_index("s") * NC + lax.axis_index("c")
    base = wid * BPW
    pltpu.sync_copy(idx_hbm.at[pl.ds(base, BPW)], idx_v)
    pltpu.async_copy(tablep_hbm.at[idx_v], rows_v, sem).wait()
    pltpu.sync_copy(rows_v, out_hbm.at[pl.ds(base, BPW)])


def _sc_gather(tablep, idx):
    mesh = plsc.VectorSubcoreMesh(core_axis_name="c", subcore_axis_name="s")
    return pl.kernel(
        _gather_body,
        mesh=mesh,
        out_type=jax.ShapeDtypeStruct((B, 128), jnp.float32),
        scratch_types=[
            pltpu.VMEM((BPW,), jnp.int32),
            pltpu.VMEM((BPW, 128), jnp.float32),
            pltpu.SemaphoreType.DMA,
        ],
        compiler_params=pltpu.CompilerParams(use_tc_tiling_on_sc=True),
    )(tablep, idx)


TILE_V = 4096
NSTEP = pl.cdiv(VOCAB, TILE_V)
TAIL_V = VOCAB - (NSTEP - 1) * TILE_V
NBUF = 3


def _mm_body(emb2_ref, w_ref, b_ref, out_hbm, scratch, sems):
    i = pl.program_id(0)
    buf = lax.rem(i, NBUF)

    @pl.when(i >= NBUF)
    def _():
        pltpu.make_async_copy(
            scratch.at[buf],
            out_hbm.at[pl.ds((i - NBUF) * TILE_V, TILE_V)],
            sems.at[buf],
        ).wait()

    w = w_ref[...].astype(jnp.bfloat16)
    emb = emb2_ref[:, :EMBD].astype(jnp.bfloat16)
    acc = lax.dot_general(
        w, emb, (((0,), (1,)), ((), ())),
        preferred_element_type=jnp.float32,
    )
    bcol = lax.dot_general(
        b_ref[...],
        jnp.ones((1, 1), jnp.float32),
        (((0,), (1,)), ((), ())),
        preferred_element_type=jnp.float32,
    )
    scratch[buf] = acc + bcol

    @pl.when(i < NSTEP - 1)
    def _():
        pltpu.make_async_copy(
            scratch.at[buf],
            out_hbm.at[pl.ds(i * TILE_V, TILE_V)],
            sems.at[buf],
        ).start()

    @pl.when(i == NSTEP - 1)
    def _():
        pltpu.make_async_copy(
            scratch.at[buf, pl.ds(0, TAIL_V)],
            out_hbm.at[pl.ds(i * TILE_V, TAIL_V)],
            sems.at[buf],
        ).start()
        for k in range(NBUF):
            step = NSTEP - NBUF + k
            kbuf = step % NBUF
            if step == NSTEP - 1:
                pltpu.make_async_copy(
                    scratch.at[kbuf, pl.ds(0, TAIL_V)],
                    out_hbm.at[pl.ds(step * TILE_V, TAIL_V)],
                    sems.at[kbuf],
                ).wait()
            else:
                pltpu.make_async_copy(
                    scratch.at[kbuf],
                    out_hbm.at[pl.ds(step * TILE_V, TILE_V)],
                    sems.at[kbuf],
                ).wait()


def _tc_matmul_t(emb2, W, brow):
    return pl.pallas_call(
        _mm_body,
        grid=(NSTEP,),
        in_specs=[
            pl.BlockSpec((B, 128), lambda i: (0, 0)),
            pl.BlockSpec((EMBD, TILE_V), lambda i: (0, i)),
            pl.BlockSpec((1, TILE_V), lambda i: (0, i)),
        ],
        out_specs=pl.BlockSpec(memory_space=pl.ANY),
        out_shape=jax.ShapeDtypeStruct((VOCAB, B), jnp.float32),
        scratch_shapes=[
            pltpu.VMEM((NBUF, TILE_V, B), jnp.float32),
            pltpu.SemaphoreType.DMA((NBUF,)),
        ],
        compiler_params=pltpu.CompilerParams(
            dimension_semantics=("arbitrary",),
        ),
    )(emb2, W, brow)


@jax.jit
def kernel(idx, table, W, b):
    idx32 = idx.astype(jnp.int32)
    tablep = _tc_bridge(table.T)
    emb2 = _sc_gather(tablep, idx32)
    out_t = _tc_matmul_t(emb2, W, b.reshape(1, VOCAB))
    return out_t.T


# speedup vs baseline: 1.1056x; 1.0403x over previous
import functools

import jax
import jax.numpy as jnp
from jax import lax
from jax.experimental import pallas as pl
from jax.experimental.pallas import tpu as pltpu
from jax.experimental.pallas import tpu_sc as plsc

VOCAB = 100000
EMBD = 64
B = 1024

NC = 2
NS = 16
NW = NC * NS
BPW = B // NW

TILE_B = 8192


def _bridge_body(tt_ref, out_ref):
    t = tt_ref[...]
    tt = lax.dot_general(
        t, jnp.eye(EMBD, dtype=jnp.float32),
        (((0,), (0,)), ((), ())),
        preferred_element_type=jnp.float32,
    )
    out_ref[...] = jnp.concatenate(
        [tt, jnp.zeros((TILE_B, 128 - EMBD), jnp.float32)], axis=1
    )


def _tc_bridge(tablet):
    return pl.pallas_call(
        _bridge_body,
        grid=(pl.cdiv(VOCAB, TILE_B),),
        in_specs=[pl.BlockSpec((EMBD, TILE_B), lambda i: (0, i))],
        out_specs=pl.BlockSpec((TILE_B, 128), lambda i: (i, 0)),
        out_shape=jax.ShapeDtypeStruct((VOCAB, 128), jnp.float32),
        compiler_params=pltpu.CompilerParams(
            dimension_semantics=("arbitrary",),
        ),
    )(tablet)


def _gather_body(tablep_hbm, idx_hbm, out_hbm, idx_v, rows_v, sem):
    wid = lax.axis_index("s") * NC + lax.axis_index("c")
    base = wid * BPW
    pltpu.sync_copy(idx_hbm.at[pl.ds(base, BPW)], idx_v)
    pltpu.async_copy(tablep_hbm.at[idx_v], rows_v, sem).wait()
    pltpu.sync_copy(rows_v, out_hbm.at[pl.ds(base, BPW)])


def _sc_gather(tablep, idx):
    mesh = plsc.VectorSubcoreMesh(core_axis_name="c", subcore_axis_name="s")
    return pl.kernel(
        _gather_body,
        mesh=mesh,
        out_type=jax.ShapeDtypeStruct((B, 128), jnp.float32),
        scratch_types=[
            pltpu.VMEM((BPW,), jnp.int32),
            pltpu.VMEM((BPW, 128), jnp.float32),
            pltpu.SemaphoreType.DMA,
        ],
        compiler_params=pltpu.CompilerParams(use_tc_tiling_on_sc=True),
    )(tablep, idx)


TILE_V = 4096
NSTEP = pl.cdiv(VOCAB, TILE_V)
TAIL_V = VOCAB - (NSTEP - 1) * TILE_V
NBUF = 3


def _mm_body(emb2_ref, w_ref, b_ref, out_hbm, scratch, sems):
    i = pl.program_id(0)
    buf = lax.rem(i, NBUF)

    @pl.when(i >= NBUF)
    def _():
        pltpu.make_async_copy(
            scratch.at[buf],
            out_hbm.at[pl.ds((i - NBUF) * TILE_V, TILE_V)],
            sems.at[buf],
        ).wait()

    w = w_ref[...].astype(jnp.bfloat16)
    emb = emb2_ref[:, :EMBD].astype(jnp.bfloat16)
    acc = lax.dot_general(
        w, emb, (((0,), (1,)), ((), ())),
        preferred_element_type=jnp.float32,
    )
    bcol = lax.dot_general(
        b_ref[...],
        jnp.ones((1, 1), jnp.float32),
        (((0,), (1,)), ((), ())),
        preferred_element_type=jnp.float32,
    )
    scratch[buf] = acc + bcol

    @pl.when(i < NSTEP - 1)
    def _():
        pltpu.make_async_copy(
            scratch.at[buf],
            out_hbm.at[pl.ds(i * TILE_V, TILE_V)],
            sems.at[buf],
        ).start()

    @pl.when(i == NSTEP - 1)
    def _():
        pltpu.make_async_copy(
            scratch.at[buf, pl.ds(0, TAIL_V)],
            out_hbm.at[pl.ds(i * TILE_V, TAIL_V)],
            sems.at[buf],
        ).start()
        for k in range(NBUF):
            step = NSTEP - NBUF + k
            kbuf = step % NBUF
            if step == NSTEP - 1:
                pltpu.make_async_copy(
                    scratch.at[kbuf, pl.ds(0, TAIL_V)],
                    out_hbm.at[pl.ds(step * TILE_V, TAIL_V)],
                    sems.at[kbuf],
                ).wait()
            else:
                pltpu.make_async_copy(
                    scratch.at[kbuf],
                    out_hbm.at[pl.ds(step * TILE_V, TILE_V)],
                    sems.at[kbuf],
                ).wait()


def _tc_matmul_t(emb2, W, brow):
    return pl.pallas_call(
        _mm_body,
        grid=(NSTEP,),
        in_specs=[
            pl.BlockSpec((B, 128), lambda i: (0, 0)),
            pl.BlockSpec((EMBD, TILE_V), lambda i: (0, i)),
            pl.BlockSpec((1, TILE_V), lambda i: (0, i)),
        ],
        out_specs=pl.BlockSpec(memory_space=pl.ANY),
        out_shape=jax.ShapeDtypeStruct((VOCAB, B), jnp.float32),
        scratch_shapes=[
            pltpu.VMEM((NBUF, TILE_V, B), jnp.float32),
            pltpu.SemaphoreType.DMA((NBUF,)),
        ],
        compiler_params=pltpu.CompilerParams(
            dimension_semantics=("arbitrary",),
        ),
    )(emb2, W, brow)


@jax.jit
def kernel(idx, table, W, b):
    idx32 = idx.astype(jnp.int32)
    tablep = _tc_bridge(table.T)
    emb2 = _sc_gather(tablep, idx32)
    out_t = _tc_matmul_t(emb2, W, b.reshape(1, VOCAB))
    return out_t.T


# speedup vs baseline: 1.1205x; 1.0134x over previous
import functools

import jax
import jax.numpy as jnp
from jax import lax
from jax.experimental import pallas as pl
from jax.experimental.pallas import tpu as pltpu
from jax.experimental.pallas import tpu_sc as plsc

VOCAB = 100000
EMBD = 64
B = 1024

NC = 2
NS = 16
NW = NC * NS
BPW = B // NW

TILE_B = 16384


def _bridge_body(tt_ref, out_ref):
    t = tt_ref[...]
    tt = lax.dot_general(
        t, jnp.eye(EMBD, dtype=jnp.float32),
        (((0,), (0,)), ((), ())),
        preferred_element_type=jnp.float32,
    )
    out_ref[...] = jnp.concatenate(
        [tt, jnp.zeros((TILE_B, 128 - EMBD), jnp.float32)], axis=1
    )


def _tc_bridge(tablet):
    return pl.pallas_call(
        _bridge_body,
        grid=(pl.cdiv(VOCAB, TILE_B),),
        in_specs=[pl.BlockSpec((EMBD, TILE_B), lambda i: (0, i))],
        out_specs=pl.BlockSpec((TILE_B, 128), lambda i: (i, 0)),
        out_shape=jax.ShapeDtypeStruct((VOCAB, 128), jnp.float32),
        compiler_params=pltpu.CompilerParams(
            dimension_semantics=("arbitrary",),
        ),
    )(tablet)


def _gather_body(tablep_hbm, idx_hbm, out_hbm, idx_v, rows_v, sem):
    wid = lax.axis_index("s") * NC + lax.axis_index("c")
    base = wid * BPW
    pltpu.sync_copy(idx_hbm.at[pl.ds(base, BPW)], idx_v)
    pltpu.async_copy(tablep_hbm.at[idx_v], rows_v, sem).wait()
    pltpu.sync_copy(rows_v, out_hbm.at[pl.ds(base, BPW)])


def _sc_gather(tablep, idx):
    mesh = plsc.VectorSubcoreMesh(core_axis_name="c", subcore_axis_name="s")
    return pl.kernel(
        _gather_body,
        mesh=mesh,
        out_type=jax.ShapeDtypeStruct((B, 128), jnp.float32),
        scratch_types=[
            pltpu.VMEM((BPW,), jnp.int32),
            pltpu.VMEM((BPW, 128), jnp.float32),
            pltpu.SemaphoreType.DMA,
        ],
        compiler_params=pltpu.CompilerParams(use_tc_tiling_on_sc=True),
    )(tablep, idx)


TILE_V = 4096
NSTEP = pl.cdiv(VOCAB, TILE_V)
TAIL_V = VOCAB - (NSTEP - 1) * TILE_V
NBUF = 3


def _mm_body(emb2_ref, w_ref, b_ref, out_hbm, scratch, sems):
    i = pl.program_id(0)
    buf = lax.rem(i, NBUF)

    @pl.when(i >= NBUF)
    def _():
        pltpu.make_async_copy(
            scratch.at[buf],
            out_hbm.at[pl.ds((i - NBUF) * TILE_V, TILE_V)],
            sems.at[buf],
        ).wait()

    w = w_ref[...].astype(jnp.bfloat16)
    emb = emb2_ref[:, :EMBD].astype(jnp.bfloat16)
    acc = lax.dot_general(
        w, emb, (((0,), (1,)), ((), ())),
        preferred_element_type=jnp.float32,
    )
    bcol = lax.dot_general(
        b_ref[...],
        jnp.ones((1, 1), jnp.float32),
        (((0,), (1,)), ((), ())),
        preferred_element_type=jnp.float32,
    )
    scratch[buf] = acc + bcol

    @pl.when(i < NSTEP - 1)
    def _():
        pltpu.make_async_copy(
            scratch.at[buf],
            out_hbm.at[pl.ds(i * TILE_V, TILE_V)],
            sems.at[buf],
        ).start()

    @pl.when(i == NSTEP - 1)
    def _():
        pltpu.make_async_copy(
            scratch.at[buf, pl.ds(0, TAIL_V)],
            out_hbm.at[pl.ds(i * TILE_V, TAIL_V)],
            sems.at[buf],
        ).start()
        for k in range(NBUF):
            step = NSTEP - NBUF + k
            kbuf = step % NBUF
            if step == NSTEP - 1:
                pltpu.make_async_copy(
                    scratch.at[kbuf, pl.ds(0, TAIL_V)],
                    out_hbm.at[pl.ds(step * TILE_V, TAIL_V)],
                    sems.at[kbuf],
                ).wait()
            else:
                pltpu.make_async_copy(
                    scratch.at[kbuf],
                    out_hbm.at[pl.ds(step * TILE_V, TILE_V)],
                    sems.at[kbuf],
                ).wait()


def _tc_matmul_t(emb2, W, brow):
    return pl.pallas_call(
        _mm_body,
        grid=(NSTEP,),
        in_specs=[
            pl.BlockSpec((B, 128), lambda i: (0, 0)),
            pl.BlockSpec((EMBD, TILE_V), lambda i: (0, i)),
            pl.BlockSpec((1, TILE_V), lambda i: (0, i)),
        ],
        out_specs=pl.BlockSpec(memory_space=pl.ANY),
        out_shape=jax.ShapeDtypeStruct((VOCAB, B), jnp.float32),
        scratch_shapes=[
            pltpu.VMEM((NBUF, TILE_V, B), jnp.float32),
            pltpu.SemaphoreType.DMA((NBUF,)),
        ],
        compiler_params=pltpu.CompilerParams(
            dimension_semantics=("arbitrary",),
        ),
    )(emb2, W, brow)


@jax.jit
def kernel(idx, table, W, b):
    idx32 = idx.astype(jnp.int32)
    tablep = _tc_bridge(table.T)
    emb2 = _sc_gather(tablep, idx32)
    out_t = _tc_matmul_t(emb2, W, b.reshape(1, VOCAB))
    return out_t.T


# speedup vs baseline: 1.1230x; 1.0022x over previous
import functools

import jax
import jax.numpy as jnp
from jax import lax
from jax.experimental import pallas as pl
from jax.experimental.pallas import tpu as pltpu
from jax.experimental.pallas import tpu_sc as plsc

VOCAB = 100000
EMBD = 64
B = 1024

NC = 2
NS = 16
NW = NC * NS
BPW = B // NW

TILE_B = 32768


def _bridge_body(tt_ref, out_ref):
    t = tt_ref[...]
    tt = lax.dot_general(
        t, jnp.eye(EMBD, dtype=jnp.float32),
        (((0,), (0,)), ((), ())),
        preferred_element_type=jnp.float32,
    )
    out_ref[...] = jnp.concatenate(
        [tt, jnp.zeros((TILE_B, 128 - EMBD), jnp.float32)], axis=1
    )


def _tc_bridge(tablet):
    return pl.pallas_call(
        _bridge_body,
        grid=(pl.cdiv(VOCAB, TILE_B),),
        in_specs=[pl.BlockSpec((EMBD, TILE_B), lambda i: (0, i))],
        out_specs=pl.BlockSpec((TILE_B, 128), lambda i: (i, 0)),
        out_shape=jax.ShapeDtypeStruct((VOCAB, 128), jnp.float32),
        compiler_params=pltpu.CompilerParams(
            dimension_semantics=("arbitrary",),
        ),
    )(tablet)


def _gather_body(tablep_hbm, idx_hbm, out_hbm, idx_v, rows_v, sem):
    wid = lax.axis_index("s") * NC + lax.axis_index("c")
    base = wid * BPW
    pltpu.sync_copy(idx_hbm.at[pl.ds(base, BPW)], idx_v)
    pltpu.async_copy(tablep_hbm.at[idx_v], rows_v, sem).wait()
    pltpu.sync_copy(rows_v, out_hbm.at[pl.ds(base, BPW)])


def _sc_gather(tablep, idx):
    mesh = plsc.VectorSubcoreMesh(core_axis_name="c", subcore_axis_name="s")
    return pl.kernel(
        _gather_body,
        mesh=mesh,
        out_type=jax.ShapeDtypeStruct((B, 128), jnp.float32),
        scratch_types=[
            pltpu.VMEM((BPW,), jnp.int32),
            pltpu.VMEM((BPW, 128), jnp.float32),
            pltpu.SemaphoreType.DMA,
        ],
        compiler_params=pltpu.CompilerParams(use_tc_tiling_on_sc=True),
    )(tablep, idx)


TILE_V = 4096
NSTEP = pl.cdiv(VOCAB, TILE_V)
TAIL_V = VOCAB - (NSTEP - 1) * TILE_V
NBUF = 3


def _mm_body(emb2_ref, w_ref, b_ref, out_hbm, scratch, sems):
    i = pl.program_id(0)
    buf = lax.rem(i, NBUF)

    @pl.when(i >= NBUF)
    def _():
        pltpu.make_async_copy(
            scratch.at[buf],
            out_hbm.at[pl.ds((i - NBUF) * TILE_V, TILE_V)],
            sems.at[buf],
        ).wait()

    w = w_ref[...].astype(jnp.bfloat16)
    emb = emb2_ref[:, :EMBD].astype(jnp.bfloat16)
    acc = lax.dot_general(
        w, emb, (((0,), (1,)), ((), ())),
        preferred_element_type=jnp.float32,
    )
    bcol = lax.dot_general(
        b_ref[...],
        jnp.ones((1, 1), jnp.float32),
        (((0,), (1,)), ((), ())),
        preferred_element_type=jnp.float32,
    )
    scratch[buf] = acc + bcol

    @pl.when(i < NSTEP - 1)
    def _():
        pltpu.make_async_copy(
            scratch.at[buf],
            out_hbm.at[pl.ds(i * TILE_V, TILE_V)],
            sems.at[buf],
        ).start()

    @pl.when(i == NSTEP - 1)
    def _():
        pltpu.make_async_copy(
            scratch.at[buf, pl.ds(0, TAIL_V)],
            out_hbm.at[pl.ds(i * TILE_V, TAIL_V)],
            sems.at[buf],
        ).start()
        for k in range(NBUF):
            step = NSTEP - NBUF + k
            kbuf = step % NBUF
            if step == NSTEP - 1:
                pltpu.make_async_copy(
                    scratch.at[kbuf, pl.ds(0, TAIL_V)],
                    out_hbm.at[pl.ds(step * TILE_V, TAIL_V)],
                    sems.at[kbuf],
                ).wait()
            else:
                pltpu.make_async_copy(
                    scratch.at[kbuf],
                    out_hbm.at[pl.ds(step * TILE_V, TILE_V)],
                    sems.at[kbuf],
                ).wait()


def _tc_matmul_t(emb2, W, brow):
    return pl.pallas_call(
        _mm_body,
        grid=(NSTEP,),
        in_specs=[
            pl.BlockSpec((B, 128), lambda i: (0, 0)),
            pl.BlockSpec((EMBD, TILE_V), lambda i: (0, i)),
            pl.BlockSpec((1, TILE_V), lambda i: (0, i)),
        ],
        out_specs=pl.BlockSpec(memory_space=pl.ANY),
        out_shape=jax.ShapeDtypeStruct((VOCAB, B), jnp.float32),
        scratch_shapes=[
            pltpu.VMEM((NBUF, TILE_V, B), jnp.float32),
            pltpu.SemaphoreType.DMA((NBUF,)),
        ],
        compiler_params=pltpu.CompilerParams(
            dimension_semantics=("arbitrary",),
        ),
    )(emb2, W, brow)


@jax.jit
def kernel(idx, table, W, b):
    idx32 = idx.astype(jnp.int32)
    tablep = _tc_bridge(table.T)
    emb2 = _sc_gather(tablep, idx32)
    out_t = _tc_matmul_t(emb2, W, b.reshape(1, VOCAB))
    return out_t.T
